# Initial kernel scaffold; baseline (speedup 1.0000x reference)
import functools

import jax
import jax.numpy as jnp
from jax import lax
from jax.experimental import pallas as pl
from jax.experimental.pallas import tpu as pltpu
from jax.experimental.pallas import tpu_sc as plsc

NC = 2
NS = 16
NW = NC * NS
ROW = 128
K = 8
CHUNK = K * ROW

N_NODES = 100000
NACC = 100016
ZR = NACC // NS
NPT = N_NODES // NS

EDGE_ALIGN = ROW * NW * K


def _pad_edges(x, epad, value):
    return jnp.pad(x, (0, epad - x.shape[0]), constant_values=value)


def _sc_gather(z_pad, idx):
    rows = idx.shape[1]
    rw = rows // NW
    nchunks = rw // K
    e_tot = rows * ROW

    def body(z_hbm, idx_hbm, out_hbm, idx_v, rows_v, sem):
        wid = lax.axis_index("s") * NC + lax.axis_index("c")
        r0 = wid * rw
        for a in (0, 1):
            def chunk(i, _, a=a):
                row0 = r0 + i * K
                pltpu.sync_copy(idx_hbm.at[a, pl.ds(row0, K)], idx_v)
                cps = [
                    pltpu.async_copy(
                        z_hbm.at[idx_v.at[b]],
                        rows_v.at[pl.ds(b * ROW, ROW)], sem)
                    for b in range(K)
                ]
                for cp in cps:
                    cp.wait()
                pltpu.sync_copy(rows_v, out_hbm.at[a, pl.ds(row0 * ROW, CHUNK)])
                return 0
            lax.fori_loop(0, nchunks, chunk, 0)

    mesh = plsc.VectorSubcoreMesh(core_axis_name="c", subcore_axis_name="s")
    return pl.kernel(
        body,
        out_type=jax.ShapeDtypeStruct((2, e_tot, 16), jnp.float32),
        mesh=mesh,
        scratch_types=[
            pltpu.VMEM((K, ROW), jnp.int32),
            pltpu.VMEM((CHUNK, 16), jnp.float32),
            pltpu.SemaphoreType.DMA,
        ],
    )(z_pad, idx)


def _sc_scatter(msg, tgt_rows, zrows):
    rows = tgt_rows.shape[0]
    rw = rows // NW
    nchunks = rw // K

    def body(msg_hbm, idx_hbm, z_hbm, out_hbm, idx_v, msg_v, acc):
        c = lax.axis_index("c")
        s = lax.axis_index("s")
        wid = s * NC + c
        pltpu.sync_copy(z_hbm, acc.at[pl.ds(s * ZR, ZR)])
        plsc.subcore_barrier()

        def chunk(i, _):
            row0 = wid * rw + i * K
            pltpu.sync_copy(idx_hbm.at[pl.ds(row0, K)], idx_v)
            pltpu.sync_copy(msg_hbm.at[pl.ds(row0 * ROW, CHUNK)], msg_v)
            for b in range(K):
                pltpu.sync_copy(
                    msg_v.at[pl.ds(b * ROW, ROW)],
                    acc.at[idx_v.at[b]], add=True)
            return 0
        lax.fori_loop(0, nchunks, chunk, 0)
        plsc.subcore_barrier()
        pltpu.sync_copy(
            acc.at[pl.ds(s * NPT, NPT)],
            out_hbm.at[c, pl.ds(s * NPT, NPT)])

    mesh = plsc.VectorSubcoreMesh(core_axis_name="c", subcore_axis_name="s")
    return pl.kernel(
        body,
        out_type=jax.ShapeDtypeStruct((2, N_NODES, 16), jnp.float32),
        mesh=mesh,
        scratch_types=[
            pltpu.VMEM((K, ROW), jnp.int32),
            pltpu.VMEM((CHUNK, 16), jnp.float32),
            pltpu.VMEM_SHARED((NACC, 16), jnp.float32),
        ],
    )(msg, tgt_rows, zrows)


def _edge_mlp_body(g_ref, wa_ref, wb_ref, wf_ref, b1_ref, w2_ref, b2_ref,
                   out_ref):
    zs = g_ref[0]
    zt = g_ref[1]
    wa = wa_ref[0]
    wb = wb_ref[0]
    wf = wf_ref[0]
    b1 = b1_ref[0]
    w2 = w2_ref[0]
    b2 = b2_ref[0]

    diff = zs[:, 0:3] - zt[:, 0:3]
    dist = jnp.sum(diff * diff, axis=1, keepdims=True)
    vs0, vs1, vs2 = zs[:, 3:4], zs[:, 4:5], zs[:, 5:6]
    vt0, vt1, vt2 = zt[:, 3:4], zt[:, 4:5], zt[:, 5:6]
    cp0 = vs1 * vt2 - vs2 * vt1
    cp1 = vs2 * vt0 - vs0 * vt2
    cp2 = vs0 * vt1 - vs1 * vt0
    acp = jnp.sqrt(cp0 * cp0 + cp1 * cp1 + cp2 * cp2)

    pre = (jnp.dot(zs, wa, preferred_element_type=jnp.float32)
           + jnp.dot(zt, wb, preferred_element_type=jnp.float32)
           + dist * wf[0:1, :] + cp0 * wf[1:2, :] + cp1 * wf[2:3, :]
           + cp2 * wf[3:4, :] + acp * wf[4:5, :] + b1)
    h = jnp.tanh(pre)
    y = jnp.dot(h, w2, preferred_element_type=jnp.float32) + b2
    gate = 0.5 * (jnp.tanh(0.5 * y[:, 16:17]) + 1.0)
    out_ref[...] = gate * y[:, 0:16]


def _edge_mlp(gath, wa, wb, wf, b1, w2, b2, split_blocks, blk):
    e_tot = gath.shape[1]
    nblk = e_tot // blk
    wmap = lambda i: (i // split_blocks, 0, 0)
    return pl.pallas_call(
        _edge_mlp_body,
        grid=(nblk,),
        in_specs=[
            pl.BlockSpec((2, blk, 16), lambda i: (0, i, 0)),
            pl.BlockSpec((1, 16, 64), wmap),
            pl.BlockSpec((1, 16, 64), wmap),
            pl.BlockSpec((1, 8, 64), wmap),
            pl.BlockSpec((1, 1, 64), wmap),
            pl.BlockSpec((1, 64, 17), wmap),
            pl.BlockSpec((1, 1, 17), wmap),
        ],
        out_specs=pl.BlockSpec((blk, 16), lambda i: (i, 0)),
        out_shape=jax.ShapeDtypeStruct((e_tot, 16), jnp.float32),
    )(gath, wa, wb, wf, b1, w2, b2)


def _node_mlp_body(z_ref, ph_ref, pw_ref, w1_ref, b1_ref, w2_ref, b2_ref,
                   out_ref):
    z = z_ref[...]
    aggh = ph_ref[0] + ph_ref[1]
    aggw = pw_ref[0] + pw_ref[1]
    inp = jnp.concatenate([z, aggh, aggw], axis=1)
    h = jnp.tanh(jnp.dot(inp, w1_ref[...], preferred_element_type=jnp.float32)
                 + b1_ref[...])
    out_ref[...] = (jnp.dot(h, w2_ref[...], preferred_element_type=jnp.float32)
                    + b2_ref[...])


def _node_mlp(z, parts_h, parts_w, w1, b1, w2, b2, blk):
    n = z.shape[0]
    nblk = n // blk
    return pl.pallas_call(
        _node_mlp_body,
        grid=(nblk,),
        in_specs=[
            pl.BlockSpec((blk, 13), lambda i: (i, 0)),
            pl.BlockSpec((2, blk, 16), lambda i: (0, i, 0)),
            pl.BlockSpec((2, blk, 16), lambda i: (0, i, 0)),
            pl.BlockSpec((45, 32), lambda i: (0, 0)),
            pl.BlockSpec((1, 32), lambda i: (0, 0)),
            pl.BlockSpec((32, 13), lambda i: (0, 0)),
            pl.BlockSpec((1, 13), lambda i: (0, 0)),
        ],
        out_specs=pl.BlockSpec((blk, 13), lambda i: (i, 0)),
        out_shape=jax.ShapeDtypeStruct((n, 13), jnp.float32),
    )(z, parts_h, parts_w, w1, b1, w2, b2)


def _prep_edge_weights(W1, b1, W2, b2, Wv1, bv1, Wv2, bv2):
    W1c = jnp.concatenate([W1, Wv1], axis=1)
    b1c = jnp.concatenate([b1, bv1])[None]
    A = jnp.zeros((16, 64), jnp.float32).at[0:13].set(W1c[0:13])
    A = A.at[0:3].add(W1c[26:29])
    Bm = jnp.zeros((16, 64), jnp.float32).at[0:13].set(W1c[13:26])
    Bm = Bm.at[0:3].add(-W1c[26:29])
    Wf = jnp.zeros((8, 64), jnp.float32).at[0].set(W1c[29])
    Wf = Wf.at[1:4].set(W1c[30:33]).at[4].set(W1c[33])
    W2c = jnp.zeros((64, 17), jnp.float32).at[0:32, 0:16].set(W2)
    W2c = W2c.at[32:64, 16].set(Wv2[:, 0])
    b2c = jnp.concatenate([b2, bv2])[None]
    return A, Bm, Wf, b1c, W2c, b2c


@jax.jit
def kernel(z_h, edge_index_h_h, edge_index_world,
           We1, be1, We2, be2, Ww1, bw1, Ww2, bw2,
           Wew1, bew1, Wew2, bew2, Www1, bww1, Www2, bww2,
           Wn1, bn1, Wn2, bn2):
    B, N, F = z_h.shape
    z = z_h[0]
    z_pad = jnp.pad(z, ((0, 0), (0, 16 - F)))

    src_hh = edge_index_h_h[0, 0].astype(jnp.int32)
    tgt_hh = edge_index_h_h[0, 1].astype(jnp.int32)
    src_w = edge_index_world[0, 0].astype(jnp.int32)
    tgt_w = edge_index_world[0, 1].astype(jnp.int32)

    e_hh, e_w = src_hh.shape[0], src_w.shape[0]
    epad_hh = -(-e_hh // EDGE_ALIGN) * EDGE_ALIGN
    epad_w = -(-e_w // EDGE_ALIGN) * EDGE_ALIGN

    src_cat = jnp.concatenate([
        _pad_edges(src_hh, epad_hh, 0), _pad_edges(src_w, epad_w, 0)])
    tgt_cat = jnp.concatenate([
        _pad_edges(tgt_hh, epad_hh, 0), _pad_edges(tgt_w, epad_w, 0)])
    idx_gather = jnp.stack([src_cat, tgt_cat]).reshape(2, -1, ROW)

    gath = _sc_gather(z_pad, idx_gather)

    wsets = [_prep_edge_weights(We1, be1, We2, be2, Ww1, bw1, Ww2, bw2),
             _prep_edge_weights(Wew1, bew1, Wew2, bew2,
                                Www1, bww1, Www2, bww2)]
    wa = jnp.stack([w[0] for w in wsets])
    wb = jnp.stack([w[1] for w in wsets])
    wf = jnp.stack([w[2] for w in wsets])
    b1s = jnp.stack([w[3] for w in wsets])
    w2s = jnp.stack([w[4] for w in wsets])
    b2s = jnp.stack([w[5] for w in wsets])

    blk = 4096
    msg = _edge_mlp(gath, wa, wb, wf, b1s, w2s, b2s, epad_hh // blk, blk)

    zrows = jnp.zeros((ZR, 16), jnp.float32)
    tgt_scat_hh = _pad_edges(tgt_hh, epad_hh, N_NODES).reshape(-1, ROW)
    tgt_scat_w = _pad_edges(tgt_w, epad_w, N_NODES).reshape(-1, ROW)
    parts_h = _sc_scatter(msg[:epad_hh], tgt_scat_hh, zrows)
    parts_w = _sc_scatter(msg[epad_hh:], tgt_scat_w, zrows)

    delta = _node_mlp(z, parts_h, parts_w, Wn1, bn1[None], Wn2, bn2[None],
                      blk=2000)
    return delta[None]


# speedup vs baseline: 2.1965x; 2.1965x over previous
import functools

import jax
import jax.numpy as jnp
from jax import lax
from jax.experimental import pallas as pl
from jax.experimental.pallas import tpu as pltpu
from jax.experimental.pallas import tpu_sc as plsc

NC = 2
NS = 16
NW = NC * NS
ROW = 128
K = 8
CHUNK = K * ROW

N_NODES = 100000
NACC = 100016
ZR = NACC // NS
NPT = N_NODES // NS

EDGE_ALIGN = ROW * NW * K


def _pad_edges(x, epad, value):
    return jnp.pad(x, (0, epad - x.shape[0]), constant_values=value)


def _sc_gather(z_pad, idx):
    rows = idx.shape[1]
    rw = rows // NW
    nchunks = rw // K
    e_tot = rows * ROW

    def body(z_hbm, idx_hbm, out_hbm, idx_v, rows_v, sem):
        wid = lax.axis_index("s") * NC + lax.axis_index("c")
        r0 = wid * rw
        for a in (0, 1):
            def chunk(i, _, a=a):
                row0 = r0 + i * K
                pltpu.sync_copy(idx_hbm.at[a, pl.ds(row0, K)], idx_v)
                cps = [
                    pltpu.async_copy(
                        z_hbm.at[idx_v.at[b]],
                        rows_v.at[pl.ds(b * ROW, ROW)], sem)
                    for b in range(K)
                ]
                for cp in cps:
                    cp.wait()
                pltpu.sync_copy(rows_v, out_hbm.at[a, pl.ds(row0 * ROW, CHUNK)])
                return 0
            lax.fori_loop(0, nchunks, chunk, 0)

    mesh = plsc.VectorSubcoreMesh(core_axis_name="c", subcore_axis_name="s")
    return pl.kernel(
        body,
        out_type=jax.ShapeDtypeStruct((2, e_tot, 16), jnp.float32),
        mesh=mesh,
        compiler_params=pltpu.CompilerParams(use_tc_tiling_on_sc=False),
        scratch_types=[
            pltpu.VMEM((K, ROW), jnp.int32),
            pltpu.VMEM((CHUNK, 16), jnp.float32),
            pltpu.SemaphoreType.DMA,
        ],
    )(z_pad, idx)


def _sc_scatter(msg, tgt_rows, zrows):
    rows = tgt_rows.shape[0]
    rw = rows // NW
    nchunks = rw // K

    def body(msg_hbm, idx_hbm, z_hbm, out_hbm, idx_v, msg_v, acc):
        c = lax.axis_index("c")
        s = lax.axis_index("s")
        wid = s * NC + c
        pltpu.sync_copy(z_hbm, acc.at[pl.ds(s * ZR, ZR)])
        plsc.subcore_barrier()

        def chunk(i, _):
            row0 = wid * rw + i * K
            pltpu.sync_copy(idx_hbm.at[pl.ds(row0, K)], idx_v)
            pltpu.sync_copy(msg_hbm.at[pl.ds(row0 * ROW, CHUNK)], msg_v)
            for b in range(K):
                pltpu.sync_copy(
                    msg_v.at[pl.ds(b * ROW, ROW)],
                    acc.at[idx_v.at[b]], add=True)
            return 0
        lax.fori_loop(0, nchunks, chunk, 0)
        plsc.subcore_barrier()
        pltpu.sync_copy(
            acc.at[pl.ds(s * NPT, NPT)],
            out_hbm.at[c, pl.ds(s * NPT, NPT)])

    mesh = plsc.VectorSubcoreMesh(core_axis_name="c", subcore_axis_name="s")
    return pl.kernel(
        body,
        out_type=jax.ShapeDtypeStruct((2, N_NODES, 16), jnp.float32),
        mesh=mesh,
        compiler_params=pltpu.CompilerParams(use_tc_tiling_on_sc=False),
        scratch_types=[
            pltpu.VMEM((K, ROW), jnp.int32),
            pltpu.VMEM((CHUNK, 16), jnp.float32),
            pltpu.VMEM_SHARED((NACC, 16), jnp.float32),
        ],
    )(msg, tgt_rows, zrows)


def _edge_mlp_body(g_ref, wa_ref, wb_ref, wf_ref, b1_ref, w2_ref, b2_ref,
                   out_ref):
    zs = g_ref[0]
    zt = g_ref[1]
    wa = wa_ref[0]
    wb = wb_ref[0]
    wf = wf_ref[0]
    b1 = b1_ref[0]
    w2 = w2_ref[0]
    b2 = b2_ref[0]

    diff = zs[:, 0:3] - zt[:, 0:3]
    dist = jnp.sum(diff * diff, axis=1, keepdims=True)
    vs0, vs1, vs2 = zs[:, 3:4], zs[:, 4:5], zs[:, 5:6]
    vt0, vt1, vt2 = zt[:, 3:4], zt[:, 4:5], zt[:, 5:6]
    cp0 = vs1 * vt2 - vs2 * vt1
    cp1 = vs2 * vt0 - vs0 * vt2
    cp2 = vs0 * vt1 - vs1 * vt0
    acp = jnp.sqrt(cp0 * cp0 + cp1 * cp1 + cp2 * cp2)

    pre = (jnp.dot(zs, wa, preferred_element_type=jnp.float32)
           + jnp.dot(zt, wb, preferred_element_type=jnp.float32)
           + dist * wf[0:1, :] + cp0 * wf[1:2, :] + cp1 * wf[2:3, :]
           + cp2 * wf[3:4, :] + acp * wf[4:5, :] + b1)
    h = jnp.tanh(pre)
    y = jnp.dot(h, w2, preferred_element_type=jnp.float32) + b2
    gate = 0.5 * (jnp.tanh(0.5 * y[:, 16:17]) + 1.0)
    out_ref[...] = gate * y[:, 0:16]


def _edge_mlp(gath, wa, wb, wf, b1, w2, b2, split_blocks, blk):
    e_tot = gath.shape[1]
    nblk = e_tot // blk
    wmap = lambda i: (i // split_blocks, 0, 0)
    return pl.pallas_call(
        _edge_mlp_body,
        grid=(nblk,),
        in_specs=[
            pl.BlockSpec((2, blk, 16), lambda i: (0, i, 0)),
            pl.BlockSpec((1, 16, 64), wmap),
            pl.BlockSpec((1, 16, 64), wmap),
            pl.BlockSpec((1, 8, 64), wmap),
            pl.BlockSpec((1, 1, 64), wmap),
            pl.BlockSpec((1, 64, 17), wmap),
            pl.BlockSpec((1, 1, 17), wmap),
        ],
        out_specs=pl.BlockSpec((blk, 16), lambda i: (i, 0)),
        out_shape=jax.ShapeDtypeStruct((e_tot, 16), jnp.float32),
    )(gath, wa, wb, wf, b1, w2, b2)


def _node_mlp_body(z_ref, ph_ref, pw_ref, w1_ref, b1_ref, w2_ref, b2_ref,
                   out_ref):
    z = z_ref[...]
    aggh = ph_ref[0] + ph_ref[1]
    aggw = pw_ref[0] + pw_ref[1]
    inp = jnp.concatenate([z, aggh, aggw], axis=1)
    h = jnp.tanh(jnp.dot(inp, w1_ref[...], preferred_element_type=jnp.float32)
                 + b1_ref[...])
    out_ref[...] = (jnp.dot(h, w2_ref[...], preferred_element_type=jnp.float32)
                    + b2_ref[...])


def _node_mlp(z, parts_h, parts_w, w1, b1, w2, b2, blk):
    n = z.shape[0]
    nblk = n // blk
    return pl.pallas_call(
        _node_mlp_body,
        grid=(nblk,),
        in_specs=[
            pl.BlockSpec((blk, 13), lambda i: (i, 0)),
            pl.BlockSpec((2, blk, 16), lambda i: (0, i, 0)),
            pl.BlockSpec((2, blk, 16), lambda i: (0, i, 0)),
            pl.BlockSpec((45, 32), lambda i: (0, 0)),
            pl.BlockSpec((1, 32), lambda i: (0, 0)),
            pl.BlockSpec((32, 13), lambda i: (0, 0)),
            pl.BlockSpec((1, 13), lambda i: (0, 0)),
        ],
        out_specs=pl.BlockSpec((blk, 13), lambda i: (i, 0)),
        out_shape=jax.ShapeDtypeStruct((n, 13), jnp.float32),
    )(z, parts_h, parts_w, w1, b1, w2, b2)


def _prep_edge_weights(W1, b1, W2, b2, Wv1, bv1, Wv2, bv2):
    W1c = jnp.concatenate([W1, Wv1], axis=1)
    b1c = jnp.concatenate([b1, bv1])[None]
    A = jnp.zeros((16, 64), jnp.float32).at[0:13].set(W1c[0:13])
    A = A.at[0:3].add(W1c[26:29])
    Bm = jnp.zeros((16, 64), jnp.float32).at[0:13].set(W1c[13:26])
    Bm = Bm.at[0:3].add(-W1c[26:29])
    Wf = jnp.zeros((8, 64), jnp.float32).at[0].set(W1c[29])
    Wf = Wf.at[1:4].set(W1c[30:33]).at[4].set(W1c[33])
    W2c = jnp.zeros((64, 17), jnp.float32).at[0:32, 0:16].set(W2)
    W2c = W2c.at[32:64, 16].set(Wv2[:, 0])
    b2c = jnp.concatenate([b2, bv2])[None]
    return A, Bm, Wf, b1c, W2c, b2c


@jax.jit
def kernel(z_h, edge_index_h_h, edge_index_world,
           We1, be1, We2, be2, Ww1, bw1, Ww2, bw2,
           Wew1, bew1, Wew2, bew2, Www1, bww1, Www2, bww2,
           Wn1, bn1, Wn2, bn2):
    B, N, F = z_h.shape
    z = z_h[0]
    z_pad = jnp.pad(z, ((0, 0), (0, 16 - F)))

    src_hh = edge_index_h_h[0, 0].astype(jnp.int32)
    tgt_hh = edge_index_h_h[0, 1].astype(jnp.int32)
    src_w = edge_index_world[0, 0].astype(jnp.int32)
    tgt_w = edge_index_world[0, 1].astype(jnp.int32)

    e_hh, e_w = src_hh.shape[0], src_w.shape[0]
    epad_hh = -(-e_hh // EDGE_ALIGN) * EDGE_ALIGN
    epad_w = -(-e_w // EDGE_ALIGN) * EDGE_ALIGN

    src_cat = jnp.concatenate([
        _pad_edges(src_hh, epad_hh, 0), _pad_edges(src_w, epad_w, 0)])
    tgt_cat = jnp.concatenate([
        _pad_edges(tgt_hh, epad_hh, 0), _pad_edges(tgt_w, epad_w, 0)])
    idx_gather = jnp.stack([src_cat, tgt_cat]).reshape(2, -1, ROW)

    gath = _sc_gather(z_pad, idx_gather)

    wsets = [_prep_edge_weights(We1, be1, We2, be2, Ww1, bw1, Ww2, bw2),
             _prep_edge_weights(Wew1, bew1, Wew2, bew2,
                                Www1, bww1, Www2, bww2)]
    wa = jnp.stack([w[0] for w in wsets])
    wb = jnp.stack([w[1] for w in wsets])
    wf = jnp.stack([w[2] for w in wsets])
    b1s = jnp.stack([w[3] for w in wsets])
    w2s = jnp.stack([w[4] for w in wsets])
    b2s = jnp.stack([w[5] for w in wsets])

    blk = 4096
    msg = _edge_mlp(gath, wa, wb, wf, b1s, w2s, b2s, epad_hh // blk, blk)

    zrows = jnp.zeros((ZR, 16), jnp.float32)
    tgt_scat_hh = _pad_edges(tgt_hh, epad_hh, N_NODES).reshape(-1, ROW)
    tgt_scat_w = _pad_edges(tgt_w, epad_w, N_NODES).reshape(-1, ROW)
    parts_h = _sc_scatter(msg[:epad_hh], tgt_scat_hh, zrows)
    parts_w = _sc_scatter(msg[epad_hh:], tgt_scat_w, zrows)

    delta = _node_mlp(z, parts_h, parts_w, Wn1, bn1[None], Wn2, bn2[None],
                      blk=2000)
    return delta[None]


# speedup vs baseline: 3.3667x; 1.5328x over previous
import functools

import jax
import jax.numpy as jnp
from jax import lax
from jax.experimental import pallas as pl
from jax.experimental.pallas import tpu as pltpu
from jax.experimental.pallas import tpu_sc as plsc

NC = 2
NS = 16
NW = NC * NS
ROW = 128
K = 8
CHUNK = K * ROW

N_NODES = 100000
NACC = 100016
ZR = NACC // NS
NPT = N_NODES // NS

EDGE_ALIGN = ROW * NW * K


def _pad_edges(x, epad, value):
    return jnp.pad(x, (0, epad - x.shape[0]), constant_values=value)


def _sc_gather(z_pad, idx):
    rows = idx.shape[1]
    rw = rows // NW
    nchunks = rw // K
    e_tot = rows * ROW

    def body(z_hbm, idx_hbm, out_hbm, idx_v, rows_v, sem):
        wid = lax.axis_index("s") * NC + lax.axis_index("c")
        r0 = wid * rw
        for a in (0, 1):
            def chunk(i, _, a=a):
                row0 = r0 + i * K
                pltpu.sync_copy(idx_hbm.at[a, pl.ds(row0, K)], idx_v)
                cps = [
                    pltpu.async_copy(
                        z_hbm.at[idx_v.at[b]],
                        rows_v.at[pl.ds(b * ROW, ROW)], sem)
                    for b in range(K)
                ]
                for cp in cps:
                    cp.wait()
                pltpu.sync_copy(rows_v, out_hbm.at[a, pl.ds(row0 * ROW, CHUNK)])
                return 0
            lax.fori_loop(0, nchunks, chunk, 0)

    mesh = plsc.VectorSubcoreMesh(core_axis_name="c", subcore_axis_name="s")
    return pl.kernel(
        body,
        out_type=jax.ShapeDtypeStruct((2, e_tot, 16), jnp.float32),
        mesh=mesh,
        compiler_params=pltpu.CompilerParams(use_tc_tiling_on_sc=False),
        scratch_types=[
            pltpu.VMEM((K, ROW), jnp.int32),
            pltpu.VMEM((CHUNK, 16), jnp.float32),
            pltpu.SemaphoreType.DMA,
        ],
    )(z_pad, idx)


def _sc_scatter(msg, tgt_rows, zrows):
    rows = tgt_rows.shape[0]
    rw = rows // NW
    nchunks = rw // K

    def body(msg_hbm, idx_hbm, z_hbm, out_hbm, idx_v, msg_v, acc):
        c = lax.axis_index("c")
        s = lax.axis_index("s")
        wid = s * NC + c
        pltpu.sync_copy(z_hbm, acc.at[pl.ds(s * ZR, ZR)])
        plsc.subcore_barrier()

        def chunk(i, _):
            row0 = wid * rw + i * K
            pltpu.sync_copy(idx_hbm.at[pl.ds(row0, K)], idx_v)
            pltpu.sync_copy(msg_hbm.at[pl.ds(row0 * ROW, CHUNK)], msg_v)
            for b in range(K):
                pltpu.sync_copy(
                    msg_v.at[pl.ds(b * ROW, ROW)],
                    acc.at[idx_v.at[b]], add=True)
            return 0
        lax.fori_loop(0, nchunks, chunk, 0)
        plsc.subcore_barrier()
        pltpu.sync_copy(
            acc.at[pl.ds(s * NPT, NPT)],
            out_hbm.at[c, pl.ds(s * NPT, NPT)])

    mesh = plsc.VectorSubcoreMesh(core_axis_name="c", subcore_axis_name="s")
    return pl.kernel(
        body,
        out_type=jax.ShapeDtypeStruct((2, N_NODES, 16), jnp.float32),
        mesh=mesh,
        compiler_params=pltpu.CompilerParams(use_tc_tiling_on_sc=False),
        scratch_types=[
            pltpu.VMEM((K, ROW), jnp.int32),
            pltpu.VMEM((CHUNK, 16), jnp.float32),
            pltpu.VMEM_SHARED((NACC, 16), jnp.float32),
        ],
    )(msg, tgt_rows, zrows)


def _edge_feat_body(g_ref, out_ref):
    zs = g_ref[0]
    zt = g_ref[1]
    lane = lax.broadcasted_iota(jnp.int32, zs.shape, 1) % 16

    def m(lo, hi):
        return jnp.where((lane >= lo) & (lane < hi),
                         jnp.float32(1), jnp.float32(0))

    def roll(x, s):
        return pltpu.roll(x, s % 128, 1)

    d = zs - zt
    dsq = d * d
    dist0 = dsq + roll(dsq, -1) + roll(dsq, -2)
    p1 = zs * roll(zt, -1)
    m1 = zs * roll(zt, 1)
    m2 = zs * roll(zt, 2)
    p2 = zs * roll(zt, -2)
    ca = p1 - roll(m1, -1)
    cb = m2 - roll(p2, 2)
    cpc = ca * m(4, 5) + cb * m(5, 6) + roll(ca, 3) * m(6, 7)
    s = cpc * cpc
    ssum = s + roll(s, -1) + roll(s, -2)
    acp = jnp.sqrt(roll(ssum, 3) * m(7, 8))
    out_ref[...] = d * m(0, 3) + roll(dist0, 3) * m(3, 4) + cpc + acp


def _edge_feat(gath_i8, blk8):
    rows = gath_i8.shape[1]
    return pl.pallas_call(
        _edge_feat_body,
        grid=(rows // blk8,),
        in_specs=[pl.BlockSpec((2, blk8, 128), lambda i: (0, i, 0))],
        out_specs=pl.BlockSpec((blk8, 128), lambda i: (i, 0)),
        out_shape=jax.ShapeDtypeStruct((rows, 128), jnp.float32),
    )(gath_i8)


def _edge_mm_body(g_ref, f_ref, wz_ref, wt_ref, wf_ref, b1_ref,
                  w2m_ref, w2g_ref, b2m_ref, b2g_ref, out_ref):
    pre = (jnp.dot(g_ref[0], wz_ref[0], preferred_element_type=jnp.float32)
           + jnp.dot(g_ref[1], wt_ref[0], preferred_element_type=jnp.float32)
           + jnp.dot(f_ref[...], wf_ref[0],
                     preferred_element_type=jnp.float32)
           + b1_ref[0])
    h = jnp.tanh(pre)
    y1 = jnp.dot(h, w2m_ref[0], preferred_element_type=jnp.float32) + b2m_ref[0]
    y2 = jnp.dot(h, w2g_ref[0], preferred_element_type=jnp.float32) + b2g_ref[0]
    out_ref[...] = y1 * (0.5 * jnp.tanh(0.5 * y2) + 0.5)


def _edge_mm(gath_i2, feats_i2, ws, split_blocks, blk2):
    rows = gath_i2.shape[1]
    nblk = rows // blk2
    wmap = lambda i: (i // split_blocks, 0, 0)
    wz, wt, wf, b1, w2m, w2g, b2m, b2g = ws
    return pl.pallas_call(
        _edge_mm_body,
        grid=(nblk,),
        in_specs=[
            pl.BlockSpec((2, blk2, 32), lambda i: (0, i, 0)),
            pl.BlockSpec((blk2, 32), lambda i: (i, 0)),
            pl.BlockSpec((1, 32, 128), wmap),
            pl.BlockSpec((1, 32, 128), wmap),
            pl.BlockSpec((1, 32, 128), wmap),
            pl.BlockSpec((1, 1, 128), wmap),
            pl.BlockSpec((1, 128, 32), wmap),
            pl.BlockSpec((1, 128, 32), wmap),
            pl.BlockSpec((1, 1, 32), wmap),
            pl.BlockSpec((1, 1, 32), wmap),
        ],
        out_specs=pl.BlockSpec((blk2, 32), lambda i: (i, 0)),
        out_shape=jax.ShapeDtypeStruct((rows, 32), jnp.float32),
    )(gath_i2, feats_i2, wz, wt, wf, b1, w2m, w2g, b2m, b2g)


def _node_mlp_body(z_ref, ph_ref, pw_ref, w1_ref, b1_ref, w2_ref, b2_ref,
                   out_ref):
    z = z_ref[...]
    aggh = ph_ref[0] + ph_ref[1]
    aggw = pw_ref[0] + pw_ref[1]
    inp = jnp.concatenate([z, aggh, aggw], axis=1)
    h = jnp.tanh(jnp.dot(inp, w1_ref[...], preferred_element_type=jnp.float32)
                 + b1_ref[...])
    out_ref[...] = (jnp.dot(h, w2_ref[...], preferred_element_type=jnp.float32)
                    + b2_ref[...])


def _node_mlp(z, parts_h, parts_w, w1, b1, w2, b2, blk):
    n = z.shape[0]
    nblk = n // blk
    return pl.pallas_call(
        _node_mlp_body,
        grid=(nblk,),
        in_specs=[
            pl.BlockSpec((blk, 13), lambda i: (i, 0)),
            pl.BlockSpec((2, blk, 16), lambda i: (0, i, 0)),
            pl.BlockSpec((2, blk, 16), lambda i: (0, i, 0)),
            pl.BlockSpec((45, 32), lambda i: (0, 0)),
            pl.BlockSpec((1, 32), lambda i: (0, 0)),
            pl.BlockSpec((32, 13), lambda i: (0, 0)),
            pl.BlockSpec((1, 13), lambda i: (0, 0)),
        ],
        out_specs=pl.BlockSpec((blk, 13), lambda i: (i, 0)),
        out_shape=jax.ShapeDtypeStruct((n, 13), jnp.float32),
    )(z, parts_h, parts_w, w1, b1, w2, b2)


def _bd2(w):
    z = jnp.zeros_like(w)
    return jnp.concatenate([jnp.concatenate([w, z], axis=1),
                            jnp.concatenate([z, w], axis=1)], axis=0)


def _prep_edge_weights(W1, b1, W2, b2, Wv1, bv1, Wv2, bv2):
    W1c = jnp.concatenate([W1, Wv1], axis=1)
    b1c = jnp.concatenate([b1, bv1])[None]
    A = jnp.zeros((16, 64), jnp.float32).at[0:13].set(W1c[0:13])
    Bm = jnp.zeros((16, 64), jnp.float32).at[0:13].set(W1c[13:26])
    Wf = jnp.zeros((16, 64), jnp.float32).at[0:3].set(W1c[26:29])
    Wf = Wf.at[3].set(W1c[29]).at[4:7].set(W1c[30:33]).at[7].set(W1c[33])
    W2m = jnp.zeros((64, 16), jnp.float32).at[0:32].set(W2)
    W2g = jnp.zeros((64, 16), jnp.float32).at[32:64].set(
        jnp.tile(Wv2, (1, 16)))
    b2m = jnp.tile(b2[None], (1, 2))
    b2g = jnp.tile(bv2[None], (1, 32))
    return (_bd2(A), _bd2(Bm), _bd2(Wf), jnp.tile(b1c, (1, 2)),
            _bd2(W2m), _bd2(W2g), b2m, b2g)


@jax.jit
def kernel(z_h, edge_index_h_h, edge_index_world,
           We1, be1, We2, be2, Ww1, bw1, Ww2, bw2,
           Wew1, bew1, Wew2, bew2, Www1, bww1, Www2, bww2,
           Wn1, bn1, Wn2, bn2):
    B, N, F = z_h.shape
    z = z_h[0]
    z_pad = jnp.pad(z, ((0, 0), (0, 16 - F)))

    src_hh = edge_index_h_h[0, 0].astype(jnp.int32)
    tgt_hh = edge_index_h_h[0, 1].astype(jnp.int32)
    src_w = edge_index_world[0, 0].astype(jnp.int32)
    tgt_w = edge_index_world[0, 1].astype(jnp.int32)

    e_hh, e_w = src_hh.shape[0], src_w.shape[0]
    epad_hh = -(-e_hh // EDGE_ALIGN) * EDGE_ALIGN
    epad_w = -(-e_w // EDGE_ALIGN) * EDGE_ALIGN

    src_cat = jnp.concatenate([
        _pad_edges(src_hh, epad_hh, 0), _pad_edges(src_w, epad_w, 0)])
    tgt_cat = jnp.concatenate([
        _pad_edges(tgt_hh, epad_hh, 0), _pad_edges(tgt_w, epad_w, 0)])
    idx_gather = jnp.stack([src_cat, tgt_cat]).reshape(2, -1, ROW)

    gath = _sc_gather(z_pad, idx_gather)
    e_cat = gath.shape[1]

    wsets = [_prep_edge_weights(We1, be1, We2, be2, Ww1, bw1, Ww2, bw2),
             _prep_edge_weights(Wew1, bew1, Wew2, bew2,
                                Www1, bww1, Www2, bww2)]
    ws = tuple(jnp.stack([w[i] for w in wsets]) for i in range(8))

    blk8, blk2 = 2048, 4096
    feats = _edge_feat(gath.reshape(2, e_cat // 8, 128), blk8)
    msg2 = _edge_mm(gath.reshape(2, e_cat // 2, 32),
                    feats.reshape(e_cat // 2, 32), ws,
                    epad_hh // (2 * blk2), blk2)
    msg = msg2.reshape(e_cat, 16)

    zrows = jnp.zeros((ZR, 16), jnp.float32)
    tgt_scat_hh = _pad_edges(tgt_hh, epad_hh, N_NODES).reshape(-1, ROW)
    tgt_scat_w = _pad_edges(tgt_w, epad_w, N_NODES).reshape(-1, ROW)
    parts_h = _sc_scatter(msg[:epad_hh], tgt_scat_hh, zrows)
    parts_w = _sc_scatter(msg[epad_hh:], tgt_scat_w, zrows)

    delta = _node_mlp(z, parts_h, parts_w, Wn1, bn1[None], Wn2, bn2[None],
                      blk=2000)
    return delta[None]


# speedup vs baseline: 6.8452x; 2.0332x over previous
import functools

import jax
import jax.numpy as jnp
from jax import lax
from jax.experimental import pallas as pl
from jax.experimental.pallas import tpu as pltpu
from jax.experimental.pallas import tpu_sc as plsc

NC = 2
NS = 16
NW = NC * NS
ROW = 128
K = 8
CHUNK = K * ROW

N_NODES = 100000
NACC = 100016
ZR = NACC // NS
NPT = N_NODES // NS

EDGE_ALIGN = ROW * NW * K


def _pad_edges(x, epad, value):
    return jnp.pad(x, (0, epad - x.shape[0]), constant_values=value)


def _sc_gather(z_pad, idx):
    rows = idx.shape[1]
    rw = rows // NW
    nchunks = rw // K
    b_tot = rows * 16

    def body(z_hbm, idx_hbm, out_hbm, idx_v, rows_v, big_v, sem):
        wid = lax.axis_index("s") * NC + lax.axis_index("c")
        r0 = wid * rw
        for a in (0, 1):
            def chunk(i, _, a=a):
                row0 = r0 + i * K
                pltpu.sync_copy(idx_hbm.at[a, pl.ds(row0, K)], idx_v)
                cps = [
                    pltpu.async_copy(
                        z_hbm.at[idx_v.at[b]],
                        rows_v.at[pl.ds(b * ROW, ROW)], sem)
                    for b in range(K)
                ]
                for cp in cps:
                    cp.wait()

                def rel(j, _):
                    for k in range(8):
                        big_v[j, pl.ds(k * 16, 16)] = rows_v[j * 8 + k, :]
                    return 0
                lax.fori_loop(0, K * 16, rel, 0)
                pltpu.sync_copy(big_v, out_hbm.at[a, pl.ds(row0 * 16, K * 16)])
                return 0
            lax.fori_loop(0, nchunks, chunk, 0)

    mesh = plsc.VectorSubcoreMesh(core_axis_name="c", subcore_axis_name="s")
    return pl.kernel(
        body,
        out_type=jax.ShapeDtypeStruct((2, b_tot, 128), jnp.float32),
        mesh=mesh,
        compiler_params=pltpu.CompilerParams(use_tc_tiling_on_sc=False),
        scratch_types=[
            pltpu.VMEM((K, ROW), jnp.int32),
            pltpu.VMEM((CHUNK, 16), jnp.float32),
            pltpu.VMEM((K * 16, 128), jnp.float32),
            pltpu.SemaphoreType.DMA,
        ],
    )(z_pad, idx)


def _sc_scatter(msg128, tgt_rows, zrows, off8):
    KS = 4
    rows = tgt_rows.shape[0]
    rw = rows // NW
    nchunks = rw // KS

    def body(msg_hbm, idx_hbm, z_hbm, out_hbm, idx_v, msg_v, big_v, acc):
        c = lax.axis_index("c")
        s = lax.axis_index("s")
        wid = s * NC + c
        pltpu.sync_copy(z_hbm, acc.at[pl.ds(s * ZR, ZR)])
        plsc.subcore_barrier()

        def chunk(i, _):
            row0 = wid * rw + i * KS
            pltpu.sync_copy(idx_hbm.at[pl.ds(row0, KS)], idx_v)
            pltpu.sync_copy(
                msg_hbm.at[pl.ds(off8 + row0 * 16, KS * 16)], big_v)

            def rel(j, _):
                for k in range(8):
                    msg_v[j * 8 + k, :] = big_v[j, pl.ds(k * 16, 16)]
                return 0
            lax.fori_loop(0, KS * 16, rel, 0)
            for b in range(KS):
                pltpu.sync_copy(
                    msg_v.at[pl.ds(b * ROW, ROW)],
                    acc.at[idx_v.at[b]], add=True)
            return 0
        lax.fori_loop(0, nchunks, chunk, 0)
        plsc.subcore_barrier()
        pltpu.sync_copy(
            acc.at[pl.ds(s * NPT, NPT)],
            out_hbm.at[c, pl.ds(s * NPT, NPT)])

    mesh = plsc.VectorSubcoreMesh(core_axis_name="c", subcore_axis_name="s")
    return pl.kernel(
        body,
        out_type=jax.ShapeDtypeStruct((2, N_NODES, 16), jnp.float32),
        mesh=mesh,
        compiler_params=pltpu.CompilerParams(use_tc_tiling_on_sc=False),
        scratch_types=[
            pltpu.VMEM((KS, ROW), jnp.int32),
            pltpu.VMEM((KS * ROW, 16), jnp.float32),
            pltpu.VMEM((KS * 16, 128), jnp.float32),
            pltpu.VMEM_SHARED((NACC, 16), jnp.float32),
        ],
    )(msg128, tgt_rows, zrows)


def _edge_body(g_ref, wz_ref, wt_ref, wf_ref, b1_ref,
               w2m_ref, w2g_ref, b2m_ref, b2g_ref, out_ref):
    zs = g_ref[0]
    zt = g_ref[1]
    lane = lax.broadcasted_iota(jnp.int32, zs.shape, 1) % 16

    def m(lo, hi):
        return jnp.where((lane >= lo) & (lane < hi),
                         jnp.float32(1), jnp.float32(0))

    def roll(x, s):
        return pltpu.roll(x, s % 128, 1)

    d = zs - zt
    dsq = d * d
    dist0 = dsq + roll(dsq, -1) + roll(dsq, -2)
    p1 = zs * roll(zt, -1)
    m1 = zs * roll(zt, 1)
    m2 = zs * roll(zt, 2)
    p2 = zs * roll(zt, -2)
    ca = p1 - roll(m1, -1)
    cb = m2 - roll(p2, 2)
    cpc = ca * m(4, 5) + cb * m(5, 6) + roll(ca, 3) * m(6, 7)
    s = cpc * cpc
    ssum = s + roll(s, -1) + roll(s, -2)
    acp = jnp.sqrt(roll(ssum, 3) * m(7, 8))
    feats = d * m(0, 3) + roll(dist0, 3) * m(3, 4) + cpc + acp

    wz = wz_ref[0]
    wt = wt_ref[0]
    wf = wf_ref[0]
    b1 = b1_ref[0]
    w2m = w2m_ref[0]
    w2g = w2g_ref[0]
    b2m = b2m_ref[0]
    b2g = b2g_ref[0]
    for p in range(4):
        sl = slice(32 * p, 32 * (p + 1))
        pre = (jnp.dot(zs[:, sl], wz, preferred_element_type=jnp.float32)
               + jnp.dot(zt[:, sl], wt, preferred_element_type=jnp.float32)
               + jnp.dot(feats[:, sl], wf,
                         preferred_element_type=jnp.float32)
               + b1)
        h = jnp.tanh(pre)
        y1 = jnp.dot(h, w2m, preferred_element_type=jnp.float32) + b2m
        y2 = jnp.dot(h, w2g, preferred_element_type=jnp.float32) + b2g
        out_ref[:, sl] = y1 * (0.5 * jnp.tanh(0.5 * y2) + 0.5)


def _edge_mlp(gath128, ws, split_blocks, blk8):
    rows = gath128.shape[1]
    nblk = rows // blk8
    wmap = lambda i: (i // split_blocks, 0, 0)
    wz, wt, wf, b1, w2m, w2g, b2m, b2g = ws
    return pl.pallas_call(
        _edge_body,
        grid=(nblk,),
        in_specs=[
            pl.BlockSpec((2, blk8, 128), lambda i: (0, i, 0)),
            pl.BlockSpec((1, 32, 128), wmap),
            pl.BlockSpec((1, 32, 128), wmap),
            pl.BlockSpec((1, 32, 128), wmap),
            pl.BlockSpec((1, 1, 128), wmap),
            pl.BlockSpec((1, 128, 32), wmap),
            pl.BlockSpec((1, 128, 32), wmap),
            pl.BlockSpec((1, 1, 32), wmap),
            pl.BlockSpec((1, 1, 32), wmap),
        ],
        out_specs=pl.BlockSpec((blk8, 128), lambda i: (i, 0)),
        out_shape=jax.ShapeDtypeStruct((rows, 128), jnp.float32),
    )(gath128, wz, wt, wf, b1, w2m, w2g, b2m, b2g)


def _node_mlp_body(z_ref, ph_ref, pw_ref, w1_ref, b1_ref, w2_ref, b2_ref,
                   out_ref):
    z = z_ref[...]
    aggh = ph_ref[0] + ph_ref[1]
    aggw = pw_ref[0] + pw_ref[1]
    inp = jnp.concatenate([z, aggh, aggw], axis=1)
    h = jnp.tanh(jnp.dot(inp, w1_ref[...], preferred_element_type=jnp.float32)
                 + b1_ref[...])
    out_ref[...] = (jnp.dot(h, w2_ref[...], preferred_element_type=jnp.float32)
                    + b2_ref[...])


def _node_mlp(z, parts_h, parts_w, w1, b1, w2, b2, blk):
    n = z.shape[0]
    nblk = n // blk
    return pl.pallas_call(
        _node_mlp_body,
        grid=(nblk,),
        in_specs=[
            pl.BlockSpec((blk, 13), lambda i: (i, 0)),
            pl.BlockSpec((2, blk, 16), lambda i: (0, i, 0)),
            pl.BlockSpec((2, blk, 16), lambda i: (0, i, 0)),
            pl.BlockSpec((45, 32), lambda i: (0, 0)),
            pl.BlockSpec((1, 32), lambda i: (0, 0)),
            pl.BlockSpec((32, 13), lambda i: (0, 0)),
            pl.BlockSpec((1, 13), lambda i: (0, 0)),
        ],
        out_specs=pl.BlockSpec((blk, 13), lambda i: (i, 0)),
        out_shape=jax.ShapeDtypeStruct((n, 13), jnp.float32),
    )(z, parts_h, parts_w, w1, b1, w2, b2)


def _bd2(w):
    z = jnp.zeros_like(w)
    return jnp.concatenate([jnp.concatenate([w, z], axis=1),
                            jnp.concatenate([z, w], axis=1)], axis=0)


def _prep_edge_weights(W1, b1, W2, b2, Wv1, bv1, Wv2, bv2):
    W1c = jnp.concatenate([W1, Wv1], axis=1)
    b1c = jnp.concatenate([b1, bv1])[None]
    A = jnp.zeros((16, 64), jnp.float32).at[0:13].set(W1c[0:13])
    Bm = jnp.zeros((16, 64), jnp.float32).at[0:13].set(W1c[13:26])
    Wf = jnp.zeros((16, 64), jnp.float32).at[0:3].set(W1c[26:29])
    Wf = Wf.at[3].set(W1c[29]).at[4:7].set(W1c[30:33]).at[7].set(W1c[33])
    W2m = jnp.zeros((64, 16), jnp.float32).at[0:32].set(W2)
    W2g = jnp.zeros((64, 16), jnp.float32).at[32:64].set(
        jnp.tile(Wv2, (1, 16)))
    b2m = jnp.tile(b2[None], (1, 2))
    b2g = jnp.tile(bv2[None], (1, 32))
    return (_bd2(A), _bd2(Bm), _bd2(Wf), jnp.tile(b1c, (1, 2)),
            _bd2(W2m), _bd2(W2g), b2m, b2g)


@jax.jit
def kernel(z_h, edge_index_h_h, edge_index_world,
           We1, be1, We2, be2, Ww1, bw1, Ww2, bw2,
           Wew1, bew1, Wew2, bew2, Www1, bww1, Www2, bww2,
           Wn1, bn1, Wn2, bn2):
    B, N, F = z_h.shape
    z = z_h[0]
    z_pad = jnp.pad(z, ((0, 0), (0, 16 - F)))

    src_hh = edge_index_h_h[0, 0].astype(jnp.int32)
    tgt_hh = edge_index_h_h[0, 1].astype(jnp.int32)
    src_w = edge_index_world[0, 0].astype(jnp.int32)
    tgt_w = edge_index_world[0, 1].astype(jnp.int32)

    e_hh, e_w = src_hh.shape[0], src_w.shape[0]
    epad_hh = -(-e_hh // EDGE_ALIGN) * EDGE_ALIGN
    epad_w = -(-e_w // EDGE_ALIGN) * EDGE_ALIGN

    src_cat = jnp.concatenate([
        _pad_edges(src_hh, epad_hh, 0), _pad_edges(src_w, epad_w, 0)])
    tgt_cat = jnp.concatenate([
        _pad_edges(tgt_hh, epad_hh, 0), _pad_edges(tgt_w, epad_w, 0)])
    idx_gather = jnp.stack([src_cat, tgt_cat]).reshape(2, -1, ROW)

    gath = _sc_gather(z_pad, idx_gather)

    wsets = [_prep_edge_weights(We1, be1, We2, be2, Ww1, bw1, Ww2, bw2),
             _prep_edge_weights(Wew1, bew1, Wew2, bew2,
                                Www1, bww1, Www2, bww2)]
    ws = tuple(jnp.stack([w[i] for w in wsets]) for i in range(8))

    blk8 = 1024
    msg = _edge_mlp(gath, ws, epad_hh // (8 * blk8), blk8)

    zrows = jnp.zeros((ZR, 16), jnp.float32)
    tgt_scat_hh = _pad_edges(tgt_hh, epad_hh, N_NODES).reshape(-1, ROW)
    tgt_scat_w = _pad_edges(tgt_w, epad_w, N_NODES).reshape(-1, ROW)
    parts_h = _sc_scatter(msg, tgt_scat_hh, zrows, 0)
    parts_w = _sc_scatter(msg, tgt_scat_w, zrows, epad_hh // 8)

    delta = _node_mlp(z, parts_h, parts_w, Wn1, bn1[None], Wn2, bn2[None],
                      blk=2000)
    return delta[None]


# speedup vs baseline: 8.7808x; 1.2828x over previous
import functools

import jax
import jax.numpy as jnp
from jax import lax
from jax.experimental import pallas as pl
from jax.experimental.pallas import tpu as pltpu
from jax.experimental.pallas import tpu_sc as plsc

NC = 2
NS = 16
NW = NC * NS
ROW = 128
K = 8
CHUNK = K * ROW

N_NODES = 100000
NACC = 100016
ZR = NACC // NS
NPT = N_NODES // NS

EDGE_ALIGN = ROW * NW * K


def _pad_edges(x, epad, value):
    return jnp.pad(x, (0, epad - x.shape[0]), constant_values=value)


def _sc_gather(z_pad, idx):
    rows = idx.shape[1]
    rw = rows // NW
    nchunks = rw // K
    b_tot = rows * 16

    def body(z_hbm, idx_hbm, out_hbm, idx_v, rows_v, big_v, sem):
        wid = lax.axis_index("s") * NC + lax.axis_index("c")
        r0 = wid * rw
        for a in (0, 1):
            def chunk(i, _, a=a):
                row0 = r0 + i * K
                pltpu.sync_copy(idx_hbm.at[a, pl.ds(row0, K)], idx_v)
                cps = [
                    pltpu.async_copy(
                        z_hbm.at[idx_v.at[b]],
                        rows_v.at[pl.ds(b * ROW, ROW)], sem)
                    for b in range(K)
                ]
                for cp in cps:
                    cp.wait()

                def rel(j, _):
                    for k in range(8):
                        big_v[j, pl.ds(k * 16, 16)] = rows_v[j * 8 + k, :]
                    return 0
                lax.fori_loop(0, K * 16, rel, 0)
                pltpu.sync_copy(big_v, out_hbm.at[a, pl.ds(row0 * 16, K * 16)])
                return 0
            lax.fori_loop(0, nchunks, chunk, 0)

    mesh = plsc.VectorSubcoreMesh(core_axis_name="c", subcore_axis_name="s")
    return pl.kernel(
        body,
        out_type=jax.ShapeDtypeStruct((2, b_tot, 128), jnp.float32),
        mesh=mesh,
        compiler_params=pltpu.CompilerParams(use_tc_tiling_on_sc=False),
        scratch_types=[
            pltpu.VMEM((K, ROW), jnp.int32),
            pltpu.VMEM((CHUNK, 16), jnp.float32),
            pltpu.VMEM((K * 16, 128), jnp.float32),
            pltpu.SemaphoreType.DMA,
        ],
    )(z_pad, idx)


def _sc_scatter(msg128, tgt_rows, zrows, off8):
    KS = 4
    rows = tgt_rows.shape[0]
    rw = rows // NW
    nchunks = rw // KS

    def body(msg_hbm, idx_hbm, z_hbm, out_hbm, idx_v, msg_v, big_v, acc):
        c = lax.axis_index("c")
        s = lax.axis_index("s")
        wid = s * NC + c
        pltpu.sync_copy(z_hbm, acc.at[pl.ds(s * ZR, ZR)])
        plsc.subcore_barrier()

        def chunk(i, _):
            row0 = wid * rw + i * KS
            pltpu.sync_copy(idx_hbm.at[pl.ds(row0, KS)], idx_v)
            pltpu.sync_copy(
                msg_hbm.at[pl.ds(off8 + row0 * 16, KS * 16)], big_v)

            def rel(j, _):
                for k in range(8):
                    msg_v[j * 8 + k, :] = big_v[j, pl.ds(k * 16, 16)]
                return 0
            lax.fori_loop(0, KS * 16, rel, 0)
            for b in range(KS):
                pltpu.sync_copy(
                    msg_v.at[pl.ds(b * ROW, ROW)],
                    acc.at[idx_v.at[b]], add=True)
            return 0
        lax.fori_loop(0, nchunks, chunk, 0)
        plsc.subcore_barrier()
        pltpu.sync_copy(
            acc.at[pl.ds(s * NPT, NPT)],
            out_hbm.at[c, pl.ds(s * NPT, NPT)])

    mesh = plsc.VectorSubcoreMesh(core_axis_name="c", subcore_axis_name="s")
    return pl.kernel(
        body,
        out_type=jax.ShapeDtypeStruct((2, N_NODES, 16), jnp.float32),
        mesh=mesh,
        compiler_params=pltpu.CompilerParams(use_tc_tiling_on_sc=False),
        scratch_types=[
            pltpu.VMEM((KS, ROW), jnp.int32),
            pltpu.VMEM((KS * ROW, 16), jnp.float32),
            pltpu.VMEM((KS * 16, 128), jnp.float32),
            pltpu.VMEM_SHARED((NACC, 16), jnp.float32),
        ],
    )(msg128, tgt_rows, zrows)


def _edge_body(g_ref, wz_ref, wt_ref, wf_ref, b1_ref,
               w2m_ref, w2g_ref, b2m_ref, b2g_ref, out_ref):
    zs = g_ref[0]
    zt = g_ref[1]
    lane = lax.broadcasted_iota(jnp.int32, zs.shape, 1) % 16

    def m(lo, hi):
        return jnp.where((lane >= lo) & (lane < hi),
                         jnp.float32(1), jnp.float32(0))

    def roll(x, s):
        return pltpu.roll(x, s % 128, 1)

    d = zs - zt
    dsq = d * d
    dist0 = dsq + roll(dsq, -1) + roll(dsq, -2)
    p1 = zs * roll(zt, -1)
    m1 = zs * roll(zt, 1)
    m2 = zs * roll(zt, 2)
    p2 = zs * roll(zt, -2)
    ca = p1 - roll(m1, -1)
    cb = m2 - roll(p2, 2)
    cpc = ca * m(4, 5) + cb * m(5, 6) + roll(ca, 3) * m(6, 7)
    s = cpc * cpc
    ssum = s + roll(s, -1) + roll(s, -2)
    acp = jnp.sqrt(roll(ssum, 3) * m(7, 8))
    feats = d * m(0, 3) + roll(dist0, 3) * m(3, 4) + cpc + acp

    wz = wz_ref[0]
    wt = wt_ref[0]
    wf = wf_ref[0]
    b1 = b1_ref[0]
    w2m = w2m_ref[0]
    w2g = w2g_ref[0]
    b2m = b2m_ref[0]
    b2g = b2g_ref[0]
    for p in range(4):
        sl = slice(32 * p, 32 * (p + 1))
        pre = (jnp.dot(zs[:, sl], wz, preferred_element_type=jnp.float32)
               + jnp.dot(zt[:, sl], wt, preferred_element_type=jnp.float32)
               + jnp.dot(feats[:, sl], wf,
                         preferred_element_type=jnp.float32)
               + b1)
        h = jnp.tanh(pre)
        y1 = jnp.dot(h, w2m, preferred_element_type=jnp.float32) + b2m
        y2 = jnp.dot(h, w2g, preferred_element_type=jnp.float32) + b2g
        out_ref[:, sl] = y1 * (0.5 * jnp.tanh(0.5 * y2) + 0.5)


def _edge_mlp(gath128, ws, split_blocks, blk8):
    rows = gath128.shape[1]
    nblk = rows // blk8
    wmap = lambda i: (i // split_blocks, 0, 0)
    wz, wt, wf, b1, w2m, w2g, b2m, b2g = ws
    return pl.pallas_call(
        _edge_body,
        grid=(nblk,),
        in_specs=[
            pl.BlockSpec((2, blk8, 128), lambda i: (0, i, 0)),
            pl.BlockSpec((1, 32, 128), wmap),
            pl.BlockSpec((1, 32, 128), wmap),
            pl.BlockSpec((1, 32, 128), wmap),
            pl.BlockSpec((1, 1, 128), wmap),
            pl.BlockSpec((1, 128, 32), wmap),
            pl.BlockSpec((1, 128, 32), wmap),
            pl.BlockSpec((1, 1, 32), wmap),
            pl.BlockSpec((1, 1, 32), wmap),
        ],
        out_specs=pl.BlockSpec((blk8, 128), lambda i: (i, 0)),
        out_shape=jax.ShapeDtypeStruct((rows, 128), jnp.float32),
    )(gath128, wz, wt, wf, b1, w2m, w2g, b2m, b2g)


def _node_mlp_body(z_ref, ph_ref, pw_ref, w1_ref, b1_ref, w2_ref, b2_ref,
                   out_ref):
    z = z_ref[...]
    aggh = ph_ref[0] + ph_ref[1]
    aggw = pw_ref[0] + pw_ref[1]
    inp = jnp.concatenate([z, aggh, aggw], axis=1)
    h = jnp.tanh(jnp.dot(inp, w1_ref[...], preferred_element_type=jnp.float32)
                 + b1_ref[...])
    out_ref[...] = (jnp.dot(h, w2_ref[...], preferred_element_type=jnp.float32)
                    + b2_ref[...])


def _node_mlp(z, parts_h, parts_w, w1, b1, w2, b2, blk):
    n = z.shape[0]
    nblk = n // blk
    return pl.pallas_call(
        _node_mlp_body,
        grid=(nblk,),
        in_specs=[
            pl.BlockSpec((blk, 13), lambda i: (i, 0)),
            pl.BlockSpec((2, blk, 16), lambda i: (0, i, 0)),
            pl.BlockSpec((2, blk, 16), lambda i: (0, i, 0)),
            pl.BlockSpec((45, 32), lambda i: (0, 0)),
            pl.BlockSpec((1, 32), lambda i: (0, 0)),
            pl.BlockSpec((32, 13), lambda i: (0, 0)),
            pl.BlockSpec((1, 13), lambda i: (0, 0)),
        ],
        out_specs=pl.BlockSpec((blk, 13), lambda i: (i, 0)),
        out_shape=jax.ShapeDtypeStruct((n, 13), jnp.float32),
    )(z, parts_h, parts_w, w1, b1, w2, b2)


def _bd2(w):
    z = jnp.zeros_like(w)
    return jnp.concatenate([jnp.concatenate([w, z], axis=1),
                            jnp.concatenate([z, w], axis=1)], axis=0)


def _prep_edge_weights(W1, b1, W2, b2, Wv1, bv1, Wv2, bv2):
    W1c = jnp.concatenate([W1, Wv1], axis=1)
    b1c = jnp.concatenate([b1, bv1])[None]
    A = jnp.zeros((16, 64), jnp.float32).at[0:13].set(W1c[0:13])
    Bm = jnp.zeros((16, 64), jnp.float32).at[0:13].set(W1c[13:26])
    Wf = jnp.zeros((16, 64), jnp.float32).at[0:3].set(W1c[26:29])
    Wf = Wf.at[3].set(W1c[29]).at[4:7].set(W1c[30:33]).at[7].set(W1c[33])
    W2m = jnp.zeros((64, 16), jnp.float32).at[0:32].set(W2)
    W2g = jnp.zeros((64, 16), jnp.float32).at[32:64].set(
        jnp.tile(Wv2, (1, 16)))
    b2m = jnp.tile(b2[None], (1, 2))
    b2g = jnp.tile(bv2[None], (1, 32))
    return (_bd2(A), _bd2(Bm), _bd2(Wf), jnp.tile(b1c, (1, 2)),
            _bd2(W2m), _bd2(W2g), b2m, b2g)


@jax.jit
def kernel(z_h, edge_index_h_h, edge_index_world,
           We1, be1, We2, be2, Ww1, bw1, Ww2, bw2,
           Wew1, bew1, Wew2, bew2, Www1, bww1, Www2, bww2,
           Wn1, bn1, Wn2, bn2):
    B, N, F = z_h.shape
    z = z_h[0]
    z_pad = jnp.pad(z, ((0, 0), (0, 16 - F)))

    src_hh = edge_index_h_h[0, 0].astype(jnp.int32)
    tgt_hh = edge_index_h_h[0, 1].astype(jnp.int32)
    src_w = edge_index_world[0, 0].astype(jnp.int32)
    tgt_w = edge_index_world[0, 1].astype(jnp.int32)

    e_hh, e_w = src_hh.shape[0], src_w.shape[0]
    epad_hh = -(-e_hh // EDGE_ALIGN) * EDGE_ALIGN
    epad_w = -(-e_w // EDGE_ALIGN) * EDGE_ALIGN

    idx_hh = jnp.stack([_pad_edges(src_hh, epad_hh, 0),
                        _pad_edges(tgt_hh, epad_hh, 0)]).reshape(2, -1, ROW)
    idx_w = jnp.stack([_pad_edges(src_w, epad_w, 0),
                       _pad_edges(tgt_w, epad_w, 0)]).reshape(2, -1, ROW)

    gath_hh = _sc_gather(z_pad, idx_hh)
    gath_w = _sc_gather(z_pad, idx_w)

    wsets = [_prep_edge_weights(We1, be1, We2, be2, Ww1, bw1, Ww2, bw2),
             _prep_edge_weights(Wew1, bew1, Wew2, bew2,
                                Www1, bww1, Www2, bww2)]
    ws_hh = tuple(w[None] for w in wsets[0])
    ws_w = tuple(w[None] for w in wsets[1])

    blk8 = 1024
    nblk_hh = gath_hh.shape[1] // blk8
    nblk_w = gath_w.shape[1] // blk8
    msg_hh = _edge_mlp(gath_hh, ws_hh, nblk_hh, blk8)
    msg_w = _edge_mlp(gath_w, ws_w, nblk_w, blk8)

    zrows = jnp.zeros((ZR, 16), jnp.float32)
    tgt_scat_hh = _pad_edges(tgt_hh, epad_hh, N_NODES).reshape(-1, ROW)
    tgt_scat_w = _pad_edges(tgt_w, epad_w, N_NODES).reshape(-1, ROW)
    parts_h = _sc_scatter(msg_hh, tgt_scat_hh, zrows, 0)
    parts_w = _sc_scatter(msg_w, tgt_scat_w, zrows, 0)

    delta = _node_mlp(z, parts_h, parts_w, Wn1, bn1[None], Wn2, bn2[None],
                      blk=2000)
    return delta[None]
